# Initial kernel scaffold; baseline (speedup 1.0000x reference)
import jax
import jax.numpy as jnp
from jax import lax
from jax.experimental import pallas as pl
from jax.experimental.pallas import tpu as pltpu

N = 20000
NUM_CLASSES = 81
MAX_INST = 100
MIN_CONF = 0.7
NMS_THR = 0.3

BLK = 2000
ROWS, COLS = 8, N // 8
OUT_PAD_R, OUT_PAD_C = 104, 128


def _refine_body(win_ref, rois_ref, probs_ref, bbox_ref,
                 y1o, x1o, y2o, x2o, clso, msko, areao):
    p = probs_ref[...]
    mx = jnp.max(p, axis=1, keepdims=True)
    iota_c = lax.broadcasted_iota(jnp.int32, (BLK, NUM_CLASSES), 1)
    ids = jnp.min(jnp.where(p == mx, iota_c, NUM_CLASSES), axis=1)
    score = mx[:, 0]

    bb = bbox_ref[...]
    lane = lax.broadcasted_iota(jnp.int32, (BLK, 4 * NUM_CLASSES), 1)
    sel_mask = (lane >> 2) == ids[:, None]
    mb = jnp.where(sel_mask, bb, 0.0)
    jj = lax.broadcasted_iota(jnp.int32, (4 * NUM_CLASSES, 128), 0)
    kk = lax.broadcasted_iota(jnp.int32, (4 * NUM_CLASSES, 128), 1)
    S = ((jj & 3) == kk).astype(jnp.float32)
    d = lax.dot_general(mb, S, (((1,), (0,)), ((), ())),
                        preferred_element_type=jnp.float32)
    d0 = d[:, 0] * 0.1
    d1 = d[:, 1] * 0.1
    d2 = d[:, 2] * 0.2
    d3 = d[:, 3] * 0.2

    r = rois_ref[...]
    ry1, rx1, ry2, rx2 = r[:, 0], r[:, 1], r[:, 2], r[:, 3]
    h = ry2 - ry1
    w = rx2 - rx1
    cy = ry1 + 0.5 * h + d0 * h
    cx = rx1 + 0.5 * w + d1 * w
    h = h * jnp.exp(d2)
    w = w * jnp.exp(d3)
    y1 = cy - 0.5 * h
    x1 = cx - 0.5 * w
    y2 = y1 + h
    x2 = x1 + w
    wy1 = win_ref[0]
    wx1 = win_ref[1]
    wy2 = win_ref[2]
    wx2 = win_ref[3]
    y1 = jnp.maximum(jnp.minimum(y1, wy2), wy1)
    x1 = jnp.maximum(jnp.minimum(x1, wx2), wx1)
    y2 = jnp.maximum(jnp.minimum(y2, wy2), wy1)
    x2 = jnp.maximum(jnp.minimum(x2, wx2), wx1)

    valid = (ids > 0) & (score >= MIN_CONF)
    masked = jnp.where(valid, score, -1.0)
    area = (y2 - y1) * (x2 - x1)

    y1o[...] = y1
    x1o[...] = x1
    y2o[...] = y2
    x2o[...] = x2
    clso[...] = ids.astype(jnp.float32)
    msko[...] = masked
    areao[...] = area


def _nms_body(y1r, x1r, y2r, x2r, clsr, mskr, arear, outr, s_ref, fi_ref):
    fi = (lax.broadcasted_iota(jnp.int32, (ROWS, COLS), 0) * COLS
          + lax.broadcasted_iota(jnp.int32, (ROWS, COLS), 1))
    fi_ref[...] = fi
    s_ref[...] = mskr[...]
    outr[...] = jnp.zeros((OUT_PAD_R, OUT_PAD_C), jnp.float32)

    y1 = y1r[...]
    x1 = x1r[...]
    y2 = y2r[...]
    x2 = x2r[...]
    cls = clsr[...]
    area = arear[...]

    def body(it, _):
        s = s_ref[...]
        m = jnp.max(s)
        flat = fi_ref[...]
        idx = jnp.min(jnp.where(s == m, flat, jnp.int32(1 << 30)))
        onehot = flat == idx
        NEG = jnp.float32(-3e38)
        y1i = jnp.max(jnp.where(onehot, y1, NEG))
        x1i = jnp.max(jnp.where(onehot, x1, NEG))
        y2i = jnp.max(jnp.where(onehot, y2, NEG))
        x2i = jnp.max(jnp.where(onehot, x2, NEG))
        ci = jnp.max(jnp.where(onehot, cls, NEG))
        yy1 = jnp.maximum(y1i, y1)
        xx1 = jnp.maximum(x1i, x1)
        yy2 = jnp.minimum(y2i, y2)
        xx2 = jnp.minimum(x2i, x2)
        inter = jnp.maximum(yy2 - yy1, 0.0) * jnp.maximum(xx2 - xx1, 0.0)
        ai = (y2i - y1i) * (x2i - x1i)
        iou = inter / (ai + area - inter + 1e-8)
        supp = ((iou > NMS_THR) & (cls == ci)) | onehot
        s_ref[...] = jnp.where(supp, -1.0, s)

        okf = jnp.where(m > 0.0, 1.0, 0.0)
        lane = lax.broadcasted_iota(jnp.int32, (1, OUT_PAD_C), 1)
        row = jnp.where(lane == 0, y1i,
              jnp.where(lane == 1, x1i,
              jnp.where(lane == 2, y2i,
              jnp.where(lane == 3, x2i,
              jnp.where(lane == 4, ci,
              jnp.where(lane == 5, m, 0.0)))))) * okf
        outr[pl.ds(it, 1), :] = row
        return 0

    lax.fori_loop(0, MAX_INST, body, 0)


@jax.jit
def kernel(rois, mrcnn_class, mrcnn_bbox, image_meta):
    image_shape = image_meta[0, 4:7]
    h = image_shape[0]
    w = image_shape[1]
    scale = jnp.stack([h, w, h, w]) - 1.0
    shift = jnp.array([0.0, 0.0, 1.0, 1.0], dtype=jnp.float32)
    window = (image_meta[0, 7:11] - shift) / scale

    rois2 = rois.reshape(N, 4)
    probs2 = mrcnn_class.reshape(N, NUM_CLASSES)
    bbox2 = mrcnn_bbox.reshape(N, 4 * NUM_CLASSES)

    nblk = N // BLK
    flat_out = jax.ShapeDtypeStruct((N,), jnp.float32)
    y1, x1, y2, x2, cls, msk, area = pl.pallas_call(
        _refine_body,
        grid=(nblk,),
        in_specs=[
            pl.BlockSpec(memory_space=pltpu.SMEM),
            pl.BlockSpec((BLK, 4), lambda i: (i, 0)),
            pl.BlockSpec((BLK, NUM_CLASSES), lambda i: (i, 0)),
            pl.BlockSpec((BLK, 4 * NUM_CLASSES), lambda i: (i, 0)),
        ],
        out_specs=[pl.BlockSpec((BLK,), lambda i: (i,))] * 7,
        out_shape=[flat_out] * 7,
    )(window, rois2, probs2, bbox2)

    shaped = [a.reshape(ROWS, COLS) for a in (y1, x1, y2, x2, cls, msk, area)]
    det = pl.pallas_call(
        _nms_body,
        out_shape=jax.ShapeDtypeStruct((OUT_PAD_R, OUT_PAD_C), jnp.float32),
        scratch_shapes=[
            pltpu.VMEM((ROWS, COLS), jnp.float32),
            pltpu.VMEM((ROWS, COLS), jnp.int32),
        ],
    )(*shaped)
    return det[:MAX_INST, :6].reshape(1, MAX_INST, 6)


# speedup vs baseline: 7.8725x; 7.8725x over previous
import jax
import jax.numpy as jnp
from jax import lax
from jax.experimental import pallas as pl
from jax.experimental.pallas import tpu as pltpu

N = 20000
NUM_CLASSES = 81
MAX_INST = 100
MIN_CONF = 0.7
NMS_THR = 0.3

BLK = 2000
ROWS, COLS = 8, N // 8
OUT_PAD_R, OUT_PAD_C = 104, 128


def _refine_body(win_ref, rois_ref, probs_ref, bbox_ref,
                 y1o, x1o, y2o, x2o, clso, msko, areao):
    p = probs_ref[...]
    mx = jnp.max(p, axis=1, keepdims=True)
    iota_c = lax.broadcasted_iota(jnp.int32, (BLK, NUM_CLASSES), 1)
    ids = jnp.min(jnp.where(p == mx, iota_c, NUM_CLASSES), axis=1)
    score = mx[:, 0]

    bb = bbox_ref[...]
    lane = lax.broadcasted_iota(jnp.int32, (BLK, 4 * NUM_CLASSES), 1)
    sel_mask = (lane >> 2) == ids[:, None]
    mb = jnp.where(sel_mask, bb, 0.0)
    jj = lax.broadcasted_iota(jnp.int32, (4 * NUM_CLASSES, 128), 0)
    kk = lax.broadcasted_iota(jnp.int32, (4 * NUM_CLASSES, 128), 1)
    S = ((jj & 3) == kk).astype(jnp.float32)
    d = lax.dot_general(mb, S, (((1,), (0,)), ((), ())),
                        preferred_element_type=jnp.float32)
    d0 = d[:, 0] * 0.1
    d1 = d[:, 1] * 0.1
    d2 = d[:, 2] * 0.2
    d3 = d[:, 3] * 0.2

    r = rois_ref[...]
    ry1, rx1, ry2, rx2 = r[:, 0], r[:, 1], r[:, 2], r[:, 3]
    h = ry2 - ry1
    w = rx2 - rx1
    cy = ry1 + 0.5 * h + d0 * h
    cx = rx1 + 0.5 * w + d1 * w
    h = h * jnp.exp(d2)
    w = w * jnp.exp(d3)
    y1 = cy - 0.5 * h
    x1 = cx - 0.5 * w
    y2 = y1 + h
    x2 = x1 + w
    wy1 = win_ref[0]
    wx1 = win_ref[1]
    wy2 = win_ref[2]
    wx2 = win_ref[3]
    y1 = jnp.maximum(jnp.minimum(y1, wy2), wy1)
    x1 = jnp.maximum(jnp.minimum(x1, wx2), wx1)
    y2 = jnp.maximum(jnp.minimum(y2, wy2), wy1)
    x2 = jnp.maximum(jnp.minimum(x2, wx2), wx1)

    valid = (ids > 0) & (score >= MIN_CONF)
    masked = jnp.where(valid, score, -1.0)
    area = (y2 - y1) * (x2 - x1)

    y1o[...] = y1.reshape(1, 1, BLK)
    x1o[...] = x1.reshape(1, 1, BLK)
    y2o[...] = y2.reshape(1, 1, BLK)
    x2o[...] = x2.reshape(1, 1, BLK)
    clso[...] = ids.astype(jnp.float32).reshape(1, 1, BLK)
    msko[...] = masked.reshape(1, 1, BLK)
    areao[...] = area.reshape(1, 1, BLK)


def _nms_body(y1r, x1r, y2r, x2r, clsr, mskr, arear, outr, s_ref, fi_ref):
    fi = (lax.broadcasted_iota(jnp.int32, (ROWS, COLS), 0) * COLS
          + lax.broadcasted_iota(jnp.int32, (ROWS, COLS), 1))
    fi_ref[...] = fi
    s_ref[...] = mskr[...]
    outr[...] = jnp.zeros((OUT_PAD_R, OUT_PAD_C), jnp.float32)

    y1 = y1r[...]
    x1 = x1r[...]
    y2 = y2r[...]
    x2 = x2r[...]
    cls = clsr[...]
    area = arear[...]

    def body(it, _):
        s = s_ref[...]
        m = jnp.max(s)
        flat = fi_ref[...]
        idx = jnp.min(jnp.where(s == m, flat, jnp.int32(1 << 30)))
        onehot = flat == idx
        NEG = jnp.float32(-3e38)
        y1i = jnp.max(jnp.where(onehot, y1, NEG))
        x1i = jnp.max(jnp.where(onehot, x1, NEG))
        y2i = jnp.max(jnp.where(onehot, y2, NEG))
        x2i = jnp.max(jnp.where(onehot, x2, NEG))
        ci = jnp.max(jnp.where(onehot, cls, NEG))
        yy1 = jnp.maximum(y1i, y1)
        xx1 = jnp.maximum(x1i, x1)
        yy2 = jnp.minimum(y2i, y2)
        xx2 = jnp.minimum(x2i, x2)
        inter = jnp.maximum(yy2 - yy1, 0.0) * jnp.maximum(xx2 - xx1, 0.0)
        ai = (y2i - y1i) * (x2i - x1i)
        iou = inter / (ai + area - inter + 1e-8)
        supp = ((iou > NMS_THR) & (cls == ci)) | onehot
        s_ref[...] = jnp.where(supp, -1.0, s)

        okf = jnp.where(m > 0.0, 1.0, 0.0)
        lane = lax.broadcasted_iota(jnp.int32, (1, OUT_PAD_C), 1)
        row = jnp.where(lane == 0, y1i,
              jnp.where(lane == 1, x1i,
              jnp.where(lane == 2, y2i,
              jnp.where(lane == 3, x2i,
              jnp.where(lane == 4, ci,
              jnp.where(lane == 5, m, 0.0)))))) * okf
        outr[pl.ds(it, 1), :] = row
        return 0

    lax.fori_loop(0, MAX_INST, body, 0)


@jax.jit
def kernel(rois, mrcnn_class, mrcnn_bbox, image_meta):
    image_shape = image_meta[0, 4:7]
    h = image_shape[0]
    w = image_shape[1]
    scale = jnp.stack([h, w, h, w]) - 1.0
    shift = jnp.array([0.0, 0.0, 1.0, 1.0], dtype=jnp.float32)
    window = (image_meta[0, 7:11] - shift) / scale

    rois2 = rois.reshape(N, 4)
    probs2 = mrcnn_class.reshape(N, NUM_CLASSES)
    bbox2 = mrcnn_bbox.reshape(N, 4 * NUM_CLASSES)

    nblk = N // BLK
    flat_out = jax.ShapeDtypeStruct((nblk, 1, BLK), jnp.float32)
    y1, x1, y2, x2, cls, msk, area = pl.pallas_call(
        _refine_body,
        grid=(nblk,),
        in_specs=[
            pl.BlockSpec(memory_space=pltpu.SMEM),
            pl.BlockSpec((BLK, 4), lambda i: (i, 0)),
            pl.BlockSpec((BLK, NUM_CLASSES), lambda i: (i, 0)),
            pl.BlockSpec((BLK, 4 * NUM_CLASSES), lambda i: (i, 0)),
        ],
        out_specs=[pl.BlockSpec((1, 1, BLK), lambda i: (i, 0, 0))] * 7,
        out_shape=[flat_out] * 7,
    )(window, rois2, probs2, bbox2)

    shaped = [a.reshape(ROWS, COLS) for a in (y1, x1, y2, x2, cls, msk, area)]
    det = pl.pallas_call(
        _nms_body,
        out_shape=jax.ShapeDtypeStruct((OUT_PAD_R, OUT_PAD_C), jnp.float32),
        scratch_shapes=[
            pltpu.VMEM((ROWS, COLS), jnp.float32),
            pltpu.VMEM((ROWS, COLS), jnp.int32),
        ],
    )(*shaped)
    return det[:MAX_INST, :6].reshape(1, MAX_INST, 6)
